# Initial kernel scaffold; baseline (speedup 1.0000x reference)
import jax
import jax.numpy as jnp
from jax.experimental import pallas as pl


def kernel(inputs, weight):
    raise NotImplementedError("write your pallas kernel here")


# speedup vs baseline: 1.1030x; 1.1030x over previous
import functools

import jax
import jax.numpy as jnp
from jax import lax
from jax.experimental import pallas as pl
from jax.experimental.pallas import tpu as pltpu
from jax.experimental.pallas import tpu_sc as plsc

NUM_WORKERS = 32
GROUP = 128
GROUPS_PER_STEP = 8
EMB_DIM = 32


def _build(total_idx: int, dim: int):
    groups_per_worker = total_idx // (NUM_WORKERS * GROUP)
    steps = groups_per_worker // GROUPS_PER_STEP
    rows_per_step = GROUPS_PER_STEP * GROUP
    per_worker = groups_per_worker * GROUP

    mesh = plsc.VectorSubcoreMesh(core_axis_name="c", subcore_axis_name="s")

    @functools.partial(
        pl.kernel,
        mesh=mesh,
        compiler_params=pltpu.CompilerParams(use_tc_tiling_on_sc=False),
        out_type=jax.ShapeDtypeStruct((total_idx, dim), jnp.float32),
        scratch_types=[
            pltpu.VMEM((groups_per_worker, GROUP), jnp.int32),
            pltpu.VMEM((rows_per_step, dim), jnp.float32),
            pltpu.SemaphoreType.DMA,
        ],
    )
    def k(table_hbm, idx_hbm, out_hbm, idx_v, rows_v, sem):
        wid = lax.axis_index("c") * 16 + lax.axis_index("s")
        gbase = wid * groups_per_worker
        obase = wid * per_worker
        pltpu.sync_copy(idx_hbm.at[pl.ds(gbase, groups_per_worker)], idx_v)

        def step(i, carry):
            handles = []
            for j in range(GROUPS_PER_STEP):
                handles.append(pltpu.async_copy(
                    table_hbm.at[idx_v.at[i * GROUPS_PER_STEP + j]],
                    rows_v.at[pl.ds(j * GROUP, GROUP)],
                    sem,
                ))
            for h in handles:
                h.wait()
            pltpu.sync_copy(
                rows_v,
                out_hbm.at[pl.ds(obase + i * rows_per_step, rows_per_step)],
            )
            return carry

        lax.fori_loop(0, steps, step, 0)

    return k


def kernel(inputs, weight):
    b, t = inputs.shape
    total = b * t
    idx = inputs.reshape(total // GROUP, GROUP).astype(jnp.int32)
    k = _build(total, EMB_DIM)
    out = k(weight, idx)
    return out.reshape(b, t, EMB_DIM)


# speedup vs baseline: 1.1091x; 1.0055x over previous
import functools

import jax
import jax.numpy as jnp
from jax import lax
from jax.experimental import pallas as pl
from jax.experimental.pallas import tpu as pltpu
from jax.experimental.pallas import tpu_sc as plsc

NUM_WORKERS = 32
GROUP = 128
GROUPS_PER_STEP = 10
EMB_DIM = 32


def _build(total_idx: int, dim: int):
    groups_per_worker = total_idx // (NUM_WORKERS * GROUP)
    steps = groups_per_worker // GROUPS_PER_STEP
    rps = GROUPS_PER_STEP * GROUP
    per_worker = groups_per_worker * GROUP

    mesh = plsc.VectorSubcoreMesh(core_axis_name="c", subcore_axis_name="s")

    @functools.partial(
        pl.kernel,
        mesh=mesh,
        compiler_params=pltpu.CompilerParams(use_tc_tiling_on_sc=False),
        out_type=jax.ShapeDtypeStruct((total_idx, dim), jnp.float32),
        scratch_types=[
            pltpu.VMEM((groups_per_worker, GROUP), jnp.int32),
            pltpu.VMEM((rps, dim), jnp.float32),
            pltpu.VMEM((rps, dim), jnp.float32),
            pltpu.SemaphoreType.DMA,
            pltpu.SemaphoreType.DMA,
            pltpu.SemaphoreType.DMA,
            pltpu.SemaphoreType.DMA,
        ],
    )
    def k(table_hbm, idx_hbm, out_hbm, idx_v, rows0, rows1, g0, g1, o0, o1):
        wid = lax.axis_index("c") * 16 + lax.axis_index("s")
        gbase = wid * groups_per_worker
        obase = wid * per_worker
        pltpu.sync_copy(idx_hbm.at[pl.ds(gbase, groups_per_worker)], idx_v)

        def fire_g(i, rows, sem):
            for j in range(GROUPS_PER_STEP):
                pltpu.async_copy(
                    table_hbm.at[idx_v.at[i * GROUPS_PER_STEP + j]],
                    rows.at[pl.ds(j * GROUP, GROUP)],
                    sem,
                )

        def drain_g(rows, sem):
            for j in range(GROUPS_PER_STEP):
                pltpu.make_async_copy(
                    table_hbm.at[idx_v.at[j]],
                    rows.at[pl.ds(j * GROUP, GROUP)],
                    sem,
                ).wait()

        def fire_o(i, rows, sem):
            pltpu.async_copy(rows, out_hbm.at[pl.ds(obase + i * rps, rps)], sem)

        def wait_o(rows, sem):
            pltpu.make_async_copy(rows, out_hbm.at[pl.ds(obase, rps)], sem).wait()

        fire_g(0, rows0, g0)
        fire_g(1, rows1, g1)

        def body(kk, carry):
            i = 2 * kk
            drain_g(rows0, g0)
            fire_o(i, rows0, o0)
            drain_g(rows1, g1)
            fire_o(i + 1, rows1, o1)
            wait_o(rows0, o0)
            fire_g(i + 2, rows0, g0)
            wait_o(rows1, o1)
            fire_g(i + 3, rows1, g1)
            return carry

        lax.fori_loop(0, steps // 2 - 1, body, 0)

        i_last = steps - 2
        drain_g(rows0, g0)
        fire_o(i_last, rows0, o0)
        drain_g(rows1, g1)
        fire_o(i_last + 1, rows1, o1)
        wait_o(rows0, o0)
        wait_o(rows1, o1)

    return k


def kernel(inputs, weight):
    b, t = inputs.shape
    total = b * t
    idx = inputs.reshape(total // GROUP, GROUP).astype(jnp.int32)
    k = _build(total, EMB_DIM)
    out = k(weight, idx)
    return out.reshape(b, t, EMB_DIM)


# speedup vs baseline: 1.7349x; 1.5643x over previous
import functools

import jax
import jax.numpy as jnp
from jax import lax
from jax.experimental import pallas as pl
from jax.experimental.pallas import tpu as pltpu
from jax.experimental.pallas import tpu_sc as plsc

NUM_WORKERS = 32
LANE = 128
EMB_DIM = 32


def _build(batch: int, seq: int, dim: int):
    b_per_w = batch // NUM_WORKERS
    blocks_per_t = b_per_w // LANE
    n_blocks = seq * blocks_per_t
    sub = dim // 8
    out_rows = batch * seq * dim // LANE

    mesh = plsc.VectorSubcoreMesh(core_axis_name="c", subcore_axis_name="s")

    @functools.partial(
        pl.kernel,
        mesh=mesh,
        compiler_params=pltpu.CompilerParams(
            use_tc_tiling_on_sc=False, needs_layout_passes=False
        ),
        out_type=jax.ShapeDtypeStruct((out_rows, LANE), jnp.float32),
        scratch_types=[
            pltpu.VMEM((seq, b_per_w), jnp.int32),
            pltpu.VMEM((LANE, dim), jnp.float32),
            pltpu.VMEM((LANE, dim), jnp.float32),
            pltpu.VMEM((dim, LANE), jnp.float32),
            pltpu.VMEM((dim, LANE), jnp.float32),
            pltpu.SemaphoreType.DMA,
            pltpu.SemaphoreType.DMA,
        ],
    )
    def k(table_hbm, idx_hbm, out_hbm, idx_v, rv0, rv1, tv0, tv1, g0, g1):
        wid = lax.axis_index("c") * 16 + lax.axis_index("s")
        w32 = wid * sub * 8
        pltpu.sync_copy(idx_hbm.at[:, pl.ds(wid * b_per_w, b_per_w)], idx_v)

        rv = (rv0, rv1)
        tv = (tv0, tv1)
        gs = (g0, g1)
        iota16 = lax.iota(jnp.int32, 16)

        def fire(t_idx, bl, buf):
            ivec = idx_v.at[t_idx, pl.ds(bl * LANE, LANE)]
            pltpu.async_copy(table_hbm.at[ivec], rv[buf], gs[buf])

        def wait_g(buf):
            pltpu.make_async_copy(
                table_hbm.at[idx_v.at[0, pl.ds(0, LANE)]], rv[buf], gs[buf]
            ).wait()

        def transpose(buf):
            src = rv[buf]
            dst = tv[buf]

            def t4(i, carry):
                for kk in range(4):
                    b = i * 4 + kk
                    cols = jnp.full((16,), 0, jnp.int32) + b
                    v_lo = src[b, pl.ds(0, 16)]
                    v_hi = src[b, pl.ds(16, 16)]
                    plsc.store_scatter(dst, [iota16, cols], v_lo)
                    plsc.store_scatter(dst, [iota16 + 16, cols], v_hi)
                return carry

            lax.fori_loop(0, LANE // 4, t4, 0)

        def write_out(t_idx, bl, buf):
            for ch in range(sub):
                base = t_idx * (sub * 8 * LANE * 8 // 8)
                base = t_idx * 4096 + ch * 1024 + w32 + bl * 8
                pltpu.sync_copy(
                    tv[buf].at[pl.ds(ch * 8, 8)],
                    out_hbm.at[pl.ds(base, 8)],
                )

        fire(0, 0, 0)
        fire(0, 1, 1)

        def tbody(t, carry):
            t_next = jnp.minimum(t + 1, seq - 1)
            for bl in range(blocks_per_t):
                buf = bl % 2
                wait_g(buf)
                transpose(buf)
                if bl + 2 < blocks_per_t:
                    fire(t, bl + 2, buf)
                else:
                    fire(t_next, bl + 2 - blocks_per_t, buf)
                write_out(t, bl, buf)
            return carry

        lax.fori_loop(0, seq, tbody, 0)
        wait_g(0)
        wait_g(1)

    return k


def kernel(inputs, weight):
    b, t = inputs.shape
    idxT = inputs.T.astype(jnp.int32)
    k = _build(b, t, EMB_DIM)
    out = k(weight, idxT)
    out5 = out.reshape(t, EMB_DIM // 8, b // LANE, 8, LANE)
    return out5.transpose(2, 4, 0, 1, 3).reshape(b, t, EMB_DIM)


# speedup vs baseline: 1.8641x; 1.0744x over previous
import functools

import jax
import jax.numpy as jnp
from jax import lax
from jax.experimental import pallas as pl
from jax.experimental.pallas import tpu as pltpu
from jax.experimental.pallas import tpu_sc as plsc

NUM_WORKERS = 32
LANE = 128
EMB_DIM = 32


def _build(batch: int, seq: int, dim: int):
    b_per_w = batch // NUM_WORKERS
    blocks_per_t = b_per_w // LANE
    n_blocks = seq * blocks_per_t
    sub = dim // 8
    out_rows = batch * seq * dim // LANE

    mesh = plsc.VectorSubcoreMesh(core_axis_name="c", subcore_axis_name="s")

    @functools.partial(
        pl.kernel,
        mesh=mesh,
        compiler_params=pltpu.CompilerParams(
            use_tc_tiling_on_sc=False, needs_layout_passes=False
        ),
        out_type=jax.ShapeDtypeStruct((out_rows, LANE), jnp.float32),
        scratch_types=[
            pltpu.VMEM((seq, b_per_w), jnp.int32),
            pltpu.VMEM((4, LANE, dim), jnp.float32),
            pltpu.VMEM((4, dim, LANE), jnp.float32),
            pltpu.SemaphoreType.DMA,
            pltpu.SemaphoreType.DMA,
            pltpu.SemaphoreType.DMA,
            pltpu.SemaphoreType.DMA,
            pltpu.SemaphoreType.DMA,
            pltpu.SemaphoreType.DMA,
            pltpu.SemaphoreType.DMA,
            pltpu.SemaphoreType.DMA,
        ],
    )
    def k(table_hbm, idx_hbm, out_hbm, idx_v, rv, tv,
          g0, g1, g2, g3, o0, o1, o2, o3):
        wid = lax.axis_index("c") * 16 + lax.axis_index("s")
        w32 = wid * sub * 8
        pltpu.sync_copy(idx_hbm.at[:, pl.ds(wid * b_per_w, b_per_w)], idx_v)

        gs = (g0, g1, g2, g3)
        os_ = (o0, o1, o2, o3)
        iota16 = lax.iota(jnp.int32, 16)

        def fire(t_idx, bl):
            ivec = idx_v.at[t_idx, pl.ds(bl * LANE, LANE)]
            pltpu.async_copy(table_hbm.at[ivec], rv.at[bl], gs[bl])

        def wait_g(bl):
            pltpu.make_async_copy(
                table_hbm.at[idx_v.at[0, pl.ds(0, LANE)]], rv.at[bl], gs[bl]
            ).wait()

        def transpose(bl):
            src = rv.at[bl]
            dst = tv.at[bl]

            def t8(i, carry):
                base = i * 8
                for kk in range(8):
                    b = base + kk
                    cols = jnp.full((16,), 0, jnp.int32) + b
                    v_lo = src[b, pl.ds(0, 16)]
                    v_hi = src[b, pl.ds(16, 16)]
                    plsc.store_scatter(dst, [iota16, cols], v_lo)
                    plsc.store_scatter(dst, [iota16 + 16, cols], v_hi)
                return carry

            lax.fori_loop(0, LANE // 8, t8, 0)

        def fire_writes(t_idx, bl):
            for ch in range(sub):
                base = t_idx * 4096 + ch * 1024 + w32 + bl * 8
                pltpu.async_copy(
                    tv.at[bl, pl.ds(ch * 8, 8)],
                    out_hbm.at[pl.ds(base, 8)],
                    os_[bl],
                )

        def wait_writes(bl):
            for ch in range(sub):
                pltpu.make_async_copy(
                    tv.at[bl, pl.ds(ch * 8, 8)],
                    out_hbm.at[pl.ds(ch * 8, 8)],
                    os_[bl],
                ).wait()

        for bl in range(blocks_per_t):
            fire(0, bl)

        def tbody(t, carry):
            t_next = jnp.minimum(t + 1, seq - 1)
            for bl in range(blocks_per_t):
                wait_g(bl)

                @pl.when(t > 0)
                def _():
                    wait_writes(bl)

                transpose(bl)
                fire(t_next, bl)
                fire_writes(t, bl)
            return carry

        lax.fori_loop(0, seq, tbody, 0)
        for bl in range(blocks_per_t):
            wait_g(bl)
            wait_writes(bl)

    return k


def kernel(inputs, weight):
    b, t = inputs.shape
    idxT = inputs.T.astype(jnp.int32)
    k = _build(b, t, EMB_DIM)
    out = k(weight, idxT)
    out5 = out.reshape(t, EMB_DIM // 8, b // LANE, 8, LANE)
    return out5.transpose(2, 4, 0, 1, 3).reshape(b, t, EMB_DIM)


# speedup vs baseline: 2.0150x; 1.0810x over previous
import functools

import jax
import jax.numpy as jnp
from jax import lax
from jax.experimental import pallas as pl
from jax.experimental.pallas import tpu as pltpu
from jax.experimental.pallas import tpu_sc as plsc

NUM_WORKERS = 32
LANE = 128
EMB_DIM = 32


def _build(batch: int, seq: int, dim: int):
    b_per_w = batch // NUM_WORKERS
    blocks_per_t = b_per_w // LANE
    n_blocks = seq * blocks_per_t
    sub = dim // 8
    out_rows = batch * seq * dim // LANE

    mesh = plsc.VectorSubcoreMesh(core_axis_name="c", subcore_axis_name="s")

    @functools.partial(
        pl.kernel,
        mesh=mesh,
        compiler_params=pltpu.CompilerParams(
            use_tc_tiling_on_sc=False, needs_layout_passes=False
        ),
        out_type=jax.ShapeDtypeStruct((out_rows, LANE), jnp.float32),
        scratch_types=[
            pltpu.VMEM((seq, b_per_w), jnp.int32),
            pltpu.VMEM((4, LANE, dim), jnp.float32),
            pltpu.VMEM((4, dim, LANE), jnp.float32),
            pltpu.SemaphoreType.DMA,
            pltpu.SemaphoreType.DMA,
            pltpu.SemaphoreType.DMA,
            pltpu.SemaphoreType.DMA,
            pltpu.SemaphoreType.DMA,
            pltpu.SemaphoreType.DMA,
            pltpu.SemaphoreType.DMA,
            pltpu.SemaphoreType.DMA,
        ],
    )
    def k(table_hbm, idx_hbm, out_hbm, idx_v, rv, tv,
          g0, g1, g2, g3, o0, o1, o2, o3):
        wid = lax.axis_index("c") * 16 + lax.axis_index("s")
        w32 = wid * sub * 8
        pltpu.sync_copy(idx_hbm.at[:, pl.ds(wid * b_per_w, b_per_w)], idx_v)

        gs = (g0, g1, g2, g3)
        os_ = (o0, o1, o2, o3)
        iota16 = lax.iota(jnp.int32, 16)

        def fire(t_idx, bl):
            ivec = idx_v.at[t_idx, pl.ds(bl * LANE, LANE)]
            pltpu.async_copy(table_hbm.at[ivec], rv.at[bl], gs[bl])

        def wait_g(bl):
            pltpu.make_async_copy(
                table_hbm.at[idx_v.at[0, pl.ds(0, LANE)]], rv.at[bl], gs[bl]
            ).wait()

        def transpose(bl):
            src = rv.at[bl]
            dst = tv.at[bl]

            @plsc.parallel_loop(0, LANE, 1, unroll=8)
            def _(b):
                cols = jnp.full((16,), 0, jnp.int32) + b
                v_lo = src[b, pl.ds(0, 16)]
                v_hi = src[b, pl.ds(16, 16)]
                plsc.store_scatter(dst, [iota16, cols], v_lo)
                plsc.store_scatter(dst, [iota16 + 16, cols], v_hi)

        def fire_writes(t_idx, bl):
            for ch in range(sub):
                base = t_idx * 4096 + ch * 1024 + w32 + bl * 8
                pltpu.async_copy(
                    tv.at[bl, pl.ds(ch * 8, 8)],
                    out_hbm.at[pl.ds(base, 8)],
                    os_[bl],
                )

        def wait_writes(bl):
            for ch in range(sub):
                pltpu.make_async_copy(
                    tv.at[bl, pl.ds(ch * 8, 8)],
                    out_hbm.at[pl.ds(ch * 8, 8)],
                    os_[bl],
                ).wait()

        for bl in range(blocks_per_t):
            fire(0, bl)

        def tbody(t, carry):
            t_next = jnp.minimum(t + 1, seq - 1)
            for bl in range(blocks_per_t):
                wait_g(bl)

                @pl.when(t > 0)
                def _():
                    wait_writes(bl)

                transpose(bl)
                fire(t_next, bl)
                fire_writes(t, bl)
            return carry

        lax.fori_loop(0, seq, tbody, 0)
        for bl in range(blocks_per_t):
            wait_g(bl)
            wait_writes(bl)

    return k


def kernel(inputs, weight):
    b, t = inputs.shape
    idxT = inputs.T.astype(jnp.int32)
    k = _build(b, t, EMB_DIM)
    out = k(weight, idxT)
    out5 = out.reshape(t, EMB_DIM // 8, b // LANE, 8, LANE)
    return out5.transpose(2, 4, 0, 1, 3).reshape(b, t, EMB_DIM)


# speedup vs baseline: 2.0203x; 1.0026x over previous
import functools

import jax
import jax.numpy as jnp
from jax import lax
from jax.experimental import pallas as pl
from jax.experimental.pallas import tpu as pltpu
from jax.experimental.pallas import tpu_sc as plsc

NUM_WORKERS = 32
LANE = 128
EMB_DIM = 32


def _build(batch: int, seq: int, dim: int):
    b_per_w = batch // NUM_WORKERS
    blocks_per_t = b_per_w // LANE
    sub = dim // 8
    n_bhi = batch // LANE

    mesh = plsc.VectorSubcoreMesh(core_axis_name="c", subcore_axis_name="s")

    @functools.partial(
        pl.kernel,
        mesh=mesh,
        compiler_params=pltpu.CompilerParams(
            use_tc_tiling_on_sc=False, needs_layout_passes=False
        ),
        out_type=jax.ShapeDtypeStruct((seq, sub, n_bhi, 8, LANE), jnp.float32),
        scratch_types=[
            pltpu.VMEM((seq, b_per_w), jnp.int32),
            pltpu.VMEM((4, LANE, dim), jnp.float32),
            pltpu.VMEM((4, sub, 8, LANE), jnp.float32),
            pltpu.SemaphoreType.DMA,
            pltpu.SemaphoreType.DMA,
            pltpu.SemaphoreType.DMA,
            pltpu.SemaphoreType.DMA,
            pltpu.SemaphoreType.DMA,
            pltpu.SemaphoreType.DMA,
            pltpu.SemaphoreType.DMA,
            pltpu.SemaphoreType.DMA,
        ],
    )
    def k(table_hbm, idx_hbm, out_hbm, idx_v, rv, tv,
          g0, g1, g2, g3, o0, o1, o2, o3):
        wid = lax.axis_index("c") * 16 + lax.axis_index("s")
        pltpu.sync_copy(idx_hbm.at[:, pl.ds(wid * b_per_w, b_per_w)], idx_v)

        gs = (g0, g1, g2, g3)
        os_ = (o0, o1, o2, o3)
        iota16 = lax.iota(jnp.int32, 16)
        chi_lo = jnp.right_shift(iota16, 3)
        chi_hi = chi_lo + 2
        clo = jnp.bitwise_and(iota16, 7)

        def fire(t_idx, bl):
            ivec = idx_v.at[t_idx, pl.ds(bl * LANE, LANE)]
            pltpu.async_copy(table_hbm.at[ivec], rv.at[bl], gs[bl])

        def wait_g(bl):
            pltpu.make_async_copy(
                table_hbm.at[idx_v.at[0, pl.ds(0, LANE)]], rv.at[bl], gs[bl]
            ).wait()

        def transpose(bl):
            src = rv.at[bl]
            dst = tv.at[bl]

            @plsc.parallel_loop(0, LANE, 1, unroll=16)
            def _(b):
                cols = jnp.full((16,), 0, jnp.int32) + b
                v_lo = src[b, pl.ds(0, 16)]
                v_hi = src[b, pl.ds(16, 16)]
                plsc.store_scatter(dst, [chi_lo, clo, cols], v_lo)
                plsc.store_scatter(dst, [chi_hi, clo, cols], v_hi)

        def fire_writes(t_idx, bl):
            pltpu.async_copy(
                tv.at[bl],
                out_hbm.at[t_idx, pl.ds(0, sub), wid * blocks_per_t + bl],
                os_[bl],
            )

        def wait_writes(bl):
            pltpu.make_async_copy(
                tv.at[bl],
                out_hbm.at[0, pl.ds(0, sub), 0],
                os_[bl],
            ).wait()

        for bl in range(blocks_per_t):
            fire(0, bl)

        def tbody(t, carry):
            t_next = jnp.minimum(t + 1, seq - 1)
            for bl in range(blocks_per_t):
                wait_g(bl)

                @pl.when(t > 0)
                def _():
                    wait_writes(bl)

                transpose(bl)
                fire(t_next, bl)
                fire_writes(t, bl)
            return carry

        lax.fori_loop(0, seq, tbody, 0)
        for bl in range(blocks_per_t):
            wait_g(bl)
            wait_writes(bl)

    return k


def kernel(inputs, weight):
    b, t = inputs.shape
    idxT = inputs.T.astype(jnp.int32)
    k = _build(b, t, EMB_DIM)
    out5 = k(weight, idxT)
    return out5.transpose(2, 4, 0, 1, 3).reshape(b, t, EMB_DIM)
